# baseline (speedup 1.0000x reference)
import jax
import jax.numpy as jnp
from jax.experimental import pallas as pl
from jax.experimental.pallas import tpu as pltpu

S = 4096
M = 1024
GE = 64
CAP = 128
BS = 256
NBLK = S // BS


def _gate_kernel(x_ref, wt_ref, cw_ref, mask_ref, loss_ref, counts_ref, me_ref):
    pid = pl.program_id(0)

    @pl.when(pid == 0)
    def _init():
        counts_ref[...] = jnp.zeros_like(counts_ref)
        me_ref[...] = jnp.zeros_like(me_ref)

    x = x_ref[...]
    wt = wt_ref[...]
    logits = jnp.dot(x, wt, preferred_element_type=jnp.float32)

    row_max = jnp.max(logits, axis=1, keepdims=True)
    p = jnp.exp(logits - row_max)
    gates = p / jnp.sum(p, axis=1, keepdims=True)

    eids = jax.lax.broadcasted_iota(jnp.int32, (BS, GE), 1)
    eidx = jnp.min(jnp.where(logits == row_max, eids, GE), axis=1, keepdims=True)
    onehot_f = (eids == eidx).astype(jnp.float32)

    r = jax.lax.broadcasted_iota(jnp.int32, (BS, BS), 0)
    c = jax.lax.broadcasted_iota(jnp.int32, (BS, BS), 1)
    ltri = (r >= c).astype(jnp.float32)
    incl = jnp.dot(ltri, onehot_f, preferred_element_type=jnp.float32)

    counts = counts_ref[...]
    loc_s = jnp.sum((incl - 1.0 + counts) * onehot_f, axis=1, keepdims=True)
    kept = loc_s < CAP
    gate_val = jnp.sum(gates * onehot_f, axis=1, keepdims=True)

    pos = eidx * CAP + loc_s.astype(jnp.int32)
    pos = jnp.where(kept, pos, -1)

    fi = jax.lax.broadcasted_iota(jnp.int32, (BS, GE * CAP), 1)
    cond = fi == pos
    cw_ref[...] = jnp.where(cond, gate_val, 0.0)
    mask_ref[...] = cond

    counts_ref[...] = counts + jnp.sum(onehot_f, axis=0, keepdims=True)
    me_ref[...] = me_ref[...] + jnp.sum(gates, axis=0, keepdims=True)

    @pl.when(pid == NBLK - 1)
    def _fin():
        loss_ref[...] = jnp.sum(
            me_ref[...] * counts_ref[...], axis=(0, 1), keepdims=True
        ) * (GE / (S * S))


def kernel(in_data, W):
    wt = W.T
    cw, mask, loss = pl.pallas_call(
        _gate_kernel,
        grid=(NBLK,),
        in_specs=[
            pl.BlockSpec((BS, M), lambda i: (i, 0)),
            pl.BlockSpec((M, GE), lambda i: (0, 0)),
        ],
        out_specs=[
            pl.BlockSpec((BS, GE * CAP), lambda i: (i, 0)),
            pl.BlockSpec((BS, GE * CAP), lambda i: (i, 0)),
            pl.BlockSpec((1, 1), lambda i: (0, 0)),
        ],
        out_shape=[
            jax.ShapeDtypeStruct((S, GE * CAP), jnp.float32),
            jax.ShapeDtypeStruct((S, GE * CAP), jnp.bool_),
            jax.ShapeDtypeStruct((1, 1), jnp.float32),
        ],
        scratch_shapes=[
            pltpu.VMEM((1, GE), jnp.float32),
            pltpu.VMEM((1, GE), jnp.float32),
        ],
    )(in_data, wt)
    return (
        cw.reshape(S, GE, CAP),
        mask.reshape(S, GE, CAP),
        loss[0, 0],
    )


# speedup vs baseline: 3.2833x; 3.2833x over previous
import jax
import jax.numpy as jnp
from jax.experimental import pallas as pl
from jax.experimental.pallas import tpu as pltpu

S = 4096
M = 1024
GE = 64
CAP = 128
BS = 256
NBLK = S // BS


def _gate_kernel(x_ref, wt_ref, cw_ref, mask_ref, loss_ref, counts_ref, me_ref):
    pid = pl.program_id(0)

    @pl.when(pid == 0)
    def _init():
        counts_ref[...] = jnp.zeros_like(counts_ref)
        me_ref[...] = jnp.zeros_like(me_ref)

    x = x_ref[...]
    wt = wt_ref[...]
    logits = jnp.dot(x, wt, preferred_element_type=jnp.float32)

    row_max = jnp.max(logits, axis=1, keepdims=True)
    p = jnp.exp(logits - row_max)
    gates = p / jnp.sum(p, axis=1, keepdims=True)

    eids = jax.lax.broadcasted_iota(jnp.int32, (BS, GE), 1)
    eidx = jnp.min(jnp.where(logits == row_max, eids, GE), axis=1, keepdims=True)
    onehot_f = (eids == eidx).astype(jnp.float32)

    r = jax.lax.broadcasted_iota(jnp.int32, (BS, BS), 0)
    c = jax.lax.broadcasted_iota(jnp.int32, (BS, BS), 1)
    ltri = (r >= c).astype(jnp.float32)
    incl = jnp.dot(ltri, onehot_f, preferred_element_type=jnp.float32)

    counts = counts_ref[...]
    loc_s = jnp.sum((incl - 1.0 + counts) * onehot_f, axis=1, keepdims=True)
    kept = loc_s < CAP
    gate_val = jnp.sum(gates * onehot_f, axis=1, keepdims=True)

    pos = eidx * CAP + loc_s.astype(jnp.int32)
    pos = jnp.where(kept, pos, -1)

    fi = jax.lax.broadcasted_iota(jnp.int32, (BS, GE, CAP), 1) * CAP + \
        jax.lax.broadcasted_iota(jnp.int32, (BS, GE, CAP), 2)
    cond = fi == pos[:, :, None]
    cw_ref[...] = jnp.where(cond, gate_val[:, :, None], 0.0)
    mask_ref[...] = cond.astype(jnp.int8)

    counts_ref[...] = counts + jnp.sum(onehot_f, axis=0, keepdims=True)
    me_ref[...] = me_ref[...] + jnp.sum(gates, axis=0, keepdims=True)

    @pl.when(pid == NBLK - 1)
    def _fin():
        loss_ref[...] = jnp.sum(
            me_ref[...] * counts_ref[...], axis=(0, 1), keepdims=True
        ) * (GE / (S * S))


def kernel(in_data, W):
    wt = W.T
    cw, mask8, loss = pl.pallas_call(
        _gate_kernel,
        grid=(NBLK,),
        in_specs=[
            pl.BlockSpec((BS, M), lambda i: (i, 0)),
            pl.BlockSpec((M, GE), lambda i: (0, 0)),
        ],
        out_specs=[
            pl.BlockSpec((BS, GE, CAP), lambda i: (i, 0, 0)),
            pl.BlockSpec((BS, GE, CAP), lambda i: (i, 0, 0)),
            pl.BlockSpec((1, 1), lambda i: (0, 0)),
        ],
        out_shape=[
            jax.ShapeDtypeStruct((S, GE, CAP), jnp.float32),
            jax.ShapeDtypeStruct((S, GE, CAP), jnp.int8),
            jax.ShapeDtypeStruct((1, 1), jnp.float32),
        ],
        scratch_shapes=[
            pltpu.VMEM((1, GE), jnp.float32),
            pltpu.VMEM((1, GE), jnp.float32),
        ],
    )(in_data, wt)
    return (cw, mask8.astype(jnp.bool_), loss[0, 0])
